# baseline (speedup 1.0000x reference)
import functools

import jax
import jax.numpy as jnp
from jax import lax
from jax.experimental import pallas as pl
from jax.experimental.pallas import tpu as pltpu
from jax.experimental.pallas import tpu_sc as plsc

RB = 32
NBUF = 3
SPLIT = 49920


def _sc_geometry():
    try:
        info = plsc.get_sparse_core_info()
        return info.num_cores, info.num_subcores
    except Exception:
        return 2, 16


@functools.lru_cache(maxsize=None)
def _make_gather(V, D, B, NC, NS):
    NW = NC * NS
    assert B % NW == 0 and (B // NW) % 8 == 0
    b_per_w = B // NW
    mesh = plsc.VectorSubcoreMesh(
        core_axis_name="c", subcore_axis_name="s",
        num_cores=NC, num_subcores=NS)

    @functools.partial(
        pl.kernel, mesh=mesh,
        out_type=jax.ShapeDtypeStruct((B, D), jnp.float32),
        scratch_types=[
            pltpu.VMEM((b_per_w,), jnp.int32),
            pltpu.VMEM((b_per_w, D), jnp.float32),
            pltpu.SemaphoreType.DMA,
        ],
        compiler_params=pltpu.CompilerParams(use_tc_tiling_on_sc=False),
    )
    def gather_kernel(table_hbm, idx_hbm, out_hbm, idx_v, rows_v, sem):
        wid = lax.axis_index("s") * NC + lax.axis_index("c")
        base = wid * b_per_w
        pltpu.sync_copy(idx_hbm.at[pl.ds(base, b_per_w)], idx_v)
        pltpu.async_copy(table_hbm.at[idx_v], rows_v, sem).wait()
        pltpu.sync_copy(rows_v, out_hbm.at[pl.ds(base, b_per_w)])

    return gather_kernel


def _make_matmul_body(B, V, D, VP):
    nt = B // RB

    def body(x_ref, w_ref, b_ref, e2_hbm, o_hbm, h2_ref, e2_ref, obuf,
             sems, esem):
        i = pl.program_id(0)

        @pl.when(i == 0)
        def _():
            pltpu.make_async_copy(e2_hbm, e2_ref, esem).start()
            pltpu.make_async_copy(e2_hbm, e2_ref, esem).wait()

        @pl.when(i == 0)
        def _():
            h = lax.dot_general(
                x_ref[...], w_ref[...], (((1,), (1,)), ((), ())),
                preferred_element_type=jnp.float32) + b_ref[...]
            z = jnp.zeros((B, D), jnp.float32)
            hl = jnp.concatenate([h, z], axis=1).reshape(nt, RB, 2 * D)
            hr = jnp.concatenate([z, h], axis=1).reshape(nt, RB, 2 * D)
            h2_ref[...] = jnp.concatenate([hl, hr], axis=1).reshape(
                2 * B, 2 * D).astype(jnp.bfloat16)

        slot = lax.rem(i, NBUF)

        def left_copy(step, s):
            return pltpu.make_async_copy(
                obuf.at[s, pl.ds(0, RB), pl.ds(0, SPLIT)],
                o_hbm.at[pl.ds(step * RB, RB), pl.ds(0, SPLIT)],
                sems.at[s, 0])

        def right_copy(step, s):
            return pltpu.make_async_copy(
                obuf.at[s, pl.ds(RB, RB), :],
                o_hbm.at[pl.ds(step * RB, RB), pl.ds(SPLIT, V - SPLIT)],
                sems.at[s, 1])

        @pl.when(i >= NBUF)
        def _():
            left_copy(i - NBUF, slot).wait()
            right_copy(i - NBUF, slot).wait()

        obuf[slot] = lax.dot_general(
            h2_ref[pl.ds(2 * RB * i, 2 * RB), :], e2_ref[...],
            (((1,), (1,)), ((), ())),
            preferred_element_type=jnp.float32)
        left_copy(i, slot).start()
        right_copy(i, slot).start()

        @pl.when(i == nt - 1)
        def _():
            for k in range(NBUF):
                step = nt - NBUF + k
                left_copy(step, step % NBUF).wait()
                right_copy(step, step % NBUF).wait()

    return body


def _tc_matmul(x, proj_weight, proj_bias, embed_weight, interpret=False):
    B, D = x.shape
    V = embed_weight.shape[0]
    VP = V - SPLIT
    a = embed_weight[:SPLIT]
    bb = embed_weight[SPLIT:]
    e2 = jnp.concatenate(
        [jnp.pad(a, ((0, VP - SPLIT), (0, 0))), bb],
        axis=1).astype(jnp.bfloat16)
    nt = B // RB
    return pl.pallas_call(
        _make_matmul_body(B, V, D, VP),
        grid=(nt,),
        in_specs=[
            pl.BlockSpec((B, D), lambda i: (0, 0)),
            pl.BlockSpec((D, D), lambda i: (0, 0)),
            pl.BlockSpec((1, D), lambda i: (0, 0)),
            pl.BlockSpec(memory_space=pltpu.MemorySpace.HBM),
        ],
        out_specs=pl.BlockSpec(memory_space=pltpu.MemorySpace.HBM),
        out_shape=jax.ShapeDtypeStruct((B, V), jnp.float32),
        scratch_shapes=[
            pltpu.VMEM((2 * B, 2 * D), jnp.bfloat16),
            pltpu.VMEM((VP, 2 * D), jnp.bfloat16),
            pltpu.VMEM((NBUF, 2 * RB, V - SPLIT), jnp.float32),
            pltpu.SemaphoreType.DMA((NBUF, 2)),
            pltpu.SemaphoreType.DMA,
        ],
        compiler_params=pltpu.CompilerParams(
            dimension_semantics=("arbitrary",)),
        interpret=interpret,
    )(x, proj_weight, proj_bias.reshape(1, D), e2)


def kernel(idx, embed_weight, proj_weight, proj_bias):
    V, D = embed_weight.shape
    B = idx.shape[0]
    NC, NS = _sc_geometry()
    x = _make_gather(V, D, B, NC, NS)(embed_weight, idx.astype(jnp.int32))
    return _tc_matmul(x, proj_weight, proj_bias, embed_weight)


# speedup vs baseline: 1.0430x; 1.0430x over previous
import functools

import jax
import jax.numpy as jnp
from jax import lax
from jax.experimental import pallas as pl
from jax.experimental.pallas import tpu as pltpu
from jax.experimental.pallas import tpu_sc as plsc

RB = 64
NBUF = 2
SPLIT = 49920


def _sc_geometry():
    try:
        info = plsc.get_sparse_core_info()
        return info.num_cores, info.num_subcores
    except Exception:
        return 2, 16


@functools.lru_cache(maxsize=None)
def _make_gather(V, D, B, NC, NS):
    NW = NC * NS
    assert B % NW == 0 and (B // NW) % 8 == 0
    b_per_w = B // NW
    mesh = plsc.VectorSubcoreMesh(
        core_axis_name="c", subcore_axis_name="s",
        num_cores=NC, num_subcores=NS)

    @functools.partial(
        pl.kernel, mesh=mesh,
        out_type=jax.ShapeDtypeStruct((B, D), jnp.float32),
        scratch_types=[
            pltpu.VMEM((b_per_w,), jnp.int32),
            pltpu.VMEM((b_per_w, D), jnp.float32),
            pltpu.SemaphoreType.DMA,
        ],
        compiler_params=pltpu.CompilerParams(use_tc_tiling_on_sc=False),
    )
    def gather_kernel(table_hbm, idx_hbm, out_hbm, idx_v, rows_v, sem):
        wid = lax.axis_index("s") * NC + lax.axis_index("c")
        base = wid * b_per_w
        pltpu.sync_copy(idx_hbm.at[pl.ds(base, b_per_w)], idx_v)
        pltpu.async_copy(table_hbm.at[idx_v], rows_v, sem).wait()
        pltpu.sync_copy(rows_v, out_hbm.at[pl.ds(base, b_per_w)])

    return gather_kernel


def _make_matmul_body(B, V, D, VP):
    nt = B // RB

    def body(x_ref, w_ref, b_ref, e2_hbm, o_hbm, h2_ref, e2_ref, obuf,
             sems, esem):
        i = pl.program_id(0)

        @pl.when(i == 0)
        def _():
            pltpu.make_async_copy(e2_hbm, e2_ref, esem).start()
            pltpu.make_async_copy(e2_hbm, e2_ref, esem).wait()

        @pl.when(i == 0)
        def _():
            h = lax.dot_general(
                x_ref[...], w_ref[...], (((1,), (1,)), ((), ())),
                preferred_element_type=jnp.float32) + b_ref[...]
            z = jnp.zeros((B, D), jnp.float32)
            hl = jnp.concatenate([h, z], axis=1).reshape(nt, RB, 2 * D)
            hr = jnp.concatenate([z, h], axis=1).reshape(nt, RB, 2 * D)
            h2_ref[...] = jnp.concatenate([hl, hr], axis=1).reshape(
                2 * B, 2 * D).astype(jnp.bfloat16)

        slot = lax.rem(i, NBUF)

        def left_copy(step, s):
            return pltpu.make_async_copy(
                obuf.at[s, pl.ds(0, RB), pl.ds(0, SPLIT)],
                o_hbm.at[pl.ds(step * RB, RB), pl.ds(0, SPLIT)],
                sems.at[s, 0])

        def right_copy(step, s):
            return pltpu.make_async_copy(
                obuf.at[s, pl.ds(RB, RB), :],
                o_hbm.at[pl.ds(step * RB, RB), pl.ds(SPLIT, V - SPLIT)],
                sems.at[s, 1])

        @pl.when(i >= NBUF)
        def _():
            left_copy(i - NBUF, slot).wait()
            right_copy(i - NBUF, slot).wait()

        obuf[slot] = lax.dot_general(
            h2_ref[pl.ds(2 * RB * i, 2 * RB), :], e2_ref[...],
            (((1,), (1,)), ((), ())),
            preferred_element_type=jnp.float32)
        left_copy(i, slot).start()
        right_copy(i, slot).start()

        @pl.when(i == nt - 1)
        def _():
            for k in range(NBUF):
                step = nt - NBUF + k
                left_copy(step, step % NBUF).wait()
                right_copy(step, step % NBUF).wait()

    return body


def _tc_matmul(x, proj_weight, proj_bias, embed_weight, interpret=False):
    B, D = x.shape
    V = embed_weight.shape[0]
    VP = V - SPLIT
    a = embed_weight[:SPLIT]
    bb = embed_weight[SPLIT:]
    e2 = jnp.concatenate(
        [jnp.pad(a, ((0, VP - SPLIT), (0, 0))), bb],
        axis=1).astype(jnp.bfloat16)
    nt = B // RB
    return pl.pallas_call(
        _make_matmul_body(B, V, D, VP),
        grid=(nt,),
        in_specs=[
            pl.BlockSpec((B, D), lambda i: (0, 0)),
            pl.BlockSpec((D, D), lambda i: (0, 0)),
            pl.BlockSpec((1, D), lambda i: (0, 0)),
            pl.BlockSpec(memory_space=pltpu.MemorySpace.HBM),
        ],
        out_specs=pl.BlockSpec(memory_space=pltpu.MemorySpace.HBM),
        out_shape=jax.ShapeDtypeStruct((B, V), jnp.float32),
        scratch_shapes=[
            pltpu.VMEM((2 * B, 2 * D), jnp.bfloat16),
            pltpu.VMEM((VP, 2 * D), jnp.bfloat16),
            pltpu.VMEM((NBUF, 2 * RB, V - SPLIT), jnp.float32),
            pltpu.SemaphoreType.DMA((NBUF, 2)),
            pltpu.SemaphoreType.DMA,
        ],
        compiler_params=pltpu.CompilerParams(
            dimension_semantics=("arbitrary",),
            vmem_limit_bytes=100 * 1024 * 1024),
        interpret=interpret,
    )(x, proj_weight, proj_bias.reshape(1, D), e2)


def kernel(idx, embed_weight, proj_weight, proj_bias):
    V, D = embed_weight.shape
    B = idx.shape[0]
    NC, NS = _sc_geometry()
    x = _make_gather(V, D, B, NC, NS)(embed_weight, idx.astype(jnp.int32))
    return _tc_matmul(x, proj_weight, proj_bias, embed_weight)
